# baseline (speedup 1.0000x reference)
import functools

import jax
import jax.numpy as jnp
from jax import lax
from jax.experimental import pallas as pl
from jax.experimental.pallas import tpu as pltpu
from jax.experimental.pallas import tpu_sc as plsc

N = 50000
D = 64
HD = 32
E = 800000
G = 128
NC = 2
NS = 16
NP = 50048
EP = 819200
GSZ = 128
NBG = EP // GSZ
GPT = NBG // NS
GPTA = NBG // (NC * NS)
GB = 40
NBUF = 4
RPT = NP // NS
BM = 1000
NBLK = N // BM


def _sca_body(dst_hbm, ones_hbm, za_hbm, degp_hbm, dsti, ones_v, accd):
    c = lax.axis_index("c")
    s = lax.axis_index("s")
    r0 = s * RPT
    pltpu.sync_copy(za_hbm.at[pl.ds(r0, RPT), :], accd.at[pl.ds(r0, RPT), :])
    pltpu.sync_copy(ones_hbm, ones_v)
    plsc.subcore_barrier()
    g0_tile = c * (NBG // NC) + s * GPTA

    def outer(k, carry):
        g0 = g0_tile + k * GB
        pltpu.sync_copy(dst_hbm.at[pl.ds(g0, GB), :], dsti)

        def inner(j, carry2):
            pltpu.sync_copy(ones_v, accd.at[dsti.at[j]], add=True)
            return carry2

        return lax.fori_loop(0, GB, inner, carry)

    lax.fori_loop(0, GPTA // GB, outer, 0)
    plsc.subcore_barrier()
    pltpu.sync_copy(accd.at[pl.ds(r0, RPT), :], degp_hbm.at[c, pl.ds(r0, RPT), :])


@functools.cache
def _get_sca():
    mesh = plsc.VectorSubcoreMesh(core_axis_name="c", subcore_axis_name="s")
    return pl.kernel(
        _sca_body,
        mesh=mesh,
        out_type=jax.ShapeDtypeStruct((NC, NP, 16), jnp.float32),
        compiler_params=pltpu.CompilerParams(use_tc_tiling_on_sc=False),
        scratch_types=[
            pltpu.VMEM((GB, GSZ), jnp.int32),
            pltpu.VMEM((GSZ, 16), jnp.float32),
            pltpu.VMEM_SHARED((NP, 16), jnp.float32),
        ],
    )


def _scb_body(y_hbm, src_hbm, dst_hbm, z_hbm, s_hbm, srci, dsti, gbuf, sem,
              acc):
    c = lax.axis_index("c")
    s = lax.axis_index("s")
    r0 = s * RPT
    pltpu.sync_copy(z_hbm.at[pl.ds(r0, RPT), :], acc.at[pl.ds(r0, RPT), :])
    plsc.subcore_barrier()
    g0_tile = s * GPT

    def outer(k, carry):
        g0 = g0_tile + k * GB
        pltpu.sync_copy(src_hbm.at[c, pl.ds(g0, GB), :], srci)
        pltpu.sync_copy(dst_hbm.at[pl.ds(g0, GB), :], dsti)
        for p in range(NBUF - 1):
            pltpu.async_copy(y_hbm.at[srci.at[p]], gbuf.at[p], sem)

        def inner(j, carry2):
            @pl.when(j + NBUF - 1 < GB)
            def _():
                pltpu.async_copy(y_hbm.at[srci.at[j + NBUF - 1]],
                                 gbuf.at[(j + NBUF - 1) % NBUF], sem)

            pltpu.make_async_copy(y_hbm.at[srci.at[j]], gbuf.at[j % NBUF],
                                  sem).wait()
            pltpu.sync_copy(gbuf.at[j % NBUF], acc.at[dsti.at[j]], add=True)
            return carry2

        return lax.fori_loop(0, GB, inner, carry)

    lax.fori_loop(0, GPT // GB, outer, 0)
    plsc.subcore_barrier()
    pltpu.sync_copy(acc.at[pl.ds(r0, RPT), :], s_hbm.at[c, pl.ds(r0, RPT), :])


@functools.cache
def _get_scb():
    mesh = plsc.VectorSubcoreMesh(core_axis_name="c", subcore_axis_name="s")
    return pl.kernel(
        _scb_body,
        mesh=mesh,
        out_type=jax.ShapeDtypeStruct((NC, NP, HD), jnp.float32),
        compiler_params=pltpu.CompilerParams(use_tc_tiling_on_sc=False),
        scratch_types=[
            pltpu.VMEM((GB, GSZ), jnp.int32),
            pltpu.VMEM((GB, GSZ), jnp.int32),
            pltpu.VMEM((NBUF, GSZ, HD), jnp.float32),
            pltpu.SemaphoreType.DMA,
            pltpu.VMEM_SHARED((NP, HD), jnp.float32),
        ],
    )


def _tc1_body(x_ref, w_ref, degp_ref, y_ref, dinv_ref):
    dp = degp_ref[...]
    deg = dp[0, :, 0] + dp[1, :, 0] + 2.0
    dinv = lax.rsqrt(deg)
    xw = jnp.dot(x_ref[...], w_ref[...], preferred_element_type=jnp.float32)
    y = xw * dinv[:, None]
    y_ref[0] = y[:, :HD]
    y_ref[1] = y[:, HD:]
    dinv_ref[...] = dinv[:, None]


_tc1 = pl.pallas_call(
    _tc1_body,
    grid=(NBLK,),
    in_specs=[
        pl.BlockSpec((BM, D), lambda i: (i, 0)),
        pl.BlockSpec((D, D), lambda i: (0, 0)),
        pl.BlockSpec((NC, BM, 16), lambda i: (0, i, 0)),
    ],
    out_specs=[
        pl.BlockSpec((NC, BM, HD), lambda i: (0, i, 0)),
        pl.BlockSpec((BM, 1), lambda i: (i, 0)),
    ],
    out_shape=[
        jax.ShapeDtypeStruct((NC, N, HD), jnp.float32),
        jax.ShapeDtypeStruct((NP, 1), jnp.float32),
    ],
)


def _tcmid_body(s_ref, y_ref, dinv_ref, b_ref, w_ref, yn_ref):
    sv = s_ref[...]
    yv = y_ref[...]
    sb = jnp.concatenate([sv[0], sv[1]], axis=1)
    yb = jnp.concatenate([yv[0], yv[1]], axis=1)
    dinv = dinv_ref[...]
    h = jnp.maximum(dinv * (sb + 2.0 * yb) + b_ref[...], 0.0)
    yn = jnp.dot(h, w_ref[...], preferred_element_type=jnp.float32) * dinv
    yn_ref[0] = yn[:, :HD]
    yn_ref[1] = yn[:, HD:]


_tcmid = pl.pallas_call(
    _tcmid_body,
    grid=(NBLK,),
    in_specs=[
        pl.BlockSpec((NC, BM, HD), lambda i: (0, i, 0)),
        pl.BlockSpec((NC, BM, HD), lambda i: (0, i, 0)),
        pl.BlockSpec((BM, 1), lambda i: (i, 0)),
        pl.BlockSpec((1, D), lambda i: (0, 0)),
        pl.BlockSpec((D, D), lambda i: (0, 0)),
    ],
    out_specs=pl.BlockSpec((NC, BM, HD), lambda i: (0, i, 0)),
    out_shape=jax.ShapeDtypeStruct((NC, N, HD), jnp.float32),
)


def _tc4_body(s_ref, y_ref, dinv_ref, b_ref, batch_ref, wout_ref, bout_ref,
              out_ref, acc_ref):
    i = pl.program_id(0)

    @pl.when(i == 0)
    def _():
        acc_ref[...] = jnp.zeros_like(acc_ref)

    sv = s_ref[...]
    yv = y_ref[...]
    sb = jnp.concatenate([sv[0], sv[1]], axis=1)
    yb = jnp.concatenate([yv[0], yv[1]], axis=1)
    dinv = dinv_ref[...]
    h = jnp.maximum(dinv * (sb + 2.0 * yb) + b_ref[...], 0.0)
    oh = (batch_ref[...] == lax.broadcasted_iota(jnp.int32, (BM, G), 1))
    acc_ref[...] += lax.dot_general(
        oh.astype(jnp.float32), h, (((0,), (0,)), ((), ())),
        preferred_element_type=jnp.float32)

    @pl.when(i == NBLK - 1)
    def _():
        out_ref[...] = (jnp.dot(acc_ref[...], wout_ref[...],
                                preferred_element_type=jnp.float32)
                        + bout_ref[...])


_tc4 = pl.pallas_call(
    _tc4_body,
    grid=(NBLK,),
    in_specs=[
        pl.BlockSpec((NC, BM, HD), lambda i: (0, i, 0)),
        pl.BlockSpec((NC, BM, HD), lambda i: (0, i, 0)),
        pl.BlockSpec((BM, 1), lambda i: (i, 0)),
        pl.BlockSpec((1, D), lambda i: (0, 0)),
        pl.BlockSpec((BM, 1), lambda i: (i, 0)),
        pl.BlockSpec((D, 1), lambda i: (0, 0)),
        pl.BlockSpec((1, 1), lambda i: (0, 0)),
    ],
    out_specs=pl.BlockSpec((G, 1), lambda i: (0, 0)),
    out_shape=jax.ShapeDtypeStruct((G, 1), jnp.float32),
    scratch_shapes=[pltpu.VMEM((G, D), jnp.float32)],
)


def kernel(x, edge_index, edge_attr, batch, W1, b1, W2, b2, Wout, bout):
    src = edge_index[0].astype(jnp.int32)
    dst = edge_index[1].astype(jnp.int32)
    pad = EP - E
    srcp = jnp.concatenate([src, jnp.zeros((pad,), jnp.int32)])
    dstp = jnp.concatenate([dst, jnp.full((pad,), N, jnp.int32)])
    src2 = jnp.stack([srcp, srcp + N]).reshape(NC, NBG, GSZ)
    dst3 = dstp.reshape(NBG, GSZ)
    zeros32 = jnp.zeros((NP, HD), jnp.float32)
    zeros16 = jnp.zeros((NP, 16), jnp.float32)
    ones16 = jnp.ones((GSZ, 16), jnp.float32)
    b1r = b1.reshape(1, D)
    b2r = b2.reshape(1, D)

    sca = _get_sca()
    scb = _get_scb()
    degp = sca(dst3, ones16, zeros16)
    y1, dinv = _tc1(x, W1, degp)
    s1 = scb(y1.reshape(NC * N, HD), src2, dst3, zeros32)
    y2 = _tcmid(s1, y1, dinv, b1r, W2)
    s2 = scb(y2.reshape(NC * N, HD), src2, dst3, zeros32)
    y3 = _tcmid(s2, y2, dinv, b2r, W2)
    s3 = scb(y3.reshape(NC * N, HD), src2, dst3, zeros32)
    return _tc4(s3, y3, dinv, b2r, batch.reshape(N, 1).astype(jnp.int32),
                Wout, bout.reshape(1, 1))


# speedup vs baseline: 1.0068x; 1.0068x over previous
import functools

import jax
import jax.numpy as jnp
from jax import lax
from jax.experimental import pallas as pl
from jax.experimental.pallas import tpu as pltpu
from jax.experimental.pallas import tpu_sc as plsc

N = 50000
D = 64
HD = 32
E = 800000
G = 128
NC = 2
NS = 16
NP = 50048
EP = 819200
GSZ = 128
NBG = EP // GSZ
GPT = NBG // NS
GPTA = NBG // (NC * NS)
GB = 40
NBUF = 5
RPT = NP // NS
BM = 1000
NBLK = N // BM


def _sca_body(dst_hbm, ones_hbm, za_hbm, degp_hbm, dsti, ones_v, accd):
    c = lax.axis_index("c")
    s = lax.axis_index("s")
    r0 = s * RPT
    pltpu.sync_copy(za_hbm.at[pl.ds(r0, RPT), :], accd.at[pl.ds(r0, RPT), :])
    pltpu.sync_copy(ones_hbm, ones_v)
    plsc.subcore_barrier()
    g0_tile = c * (NBG // NC) + s * GPTA

    def outer(k, carry):
        g0 = g0_tile + k * GB
        pltpu.sync_copy(dst_hbm.at[pl.ds(g0, GB), :], dsti)

        def inner(j, carry2):
            pltpu.sync_copy(ones_v, accd.at[dsti.at[j]], add=True)
            return carry2

        return lax.fori_loop(0, GB, inner, carry)

    lax.fori_loop(0, GPTA // GB, outer, 0)
    plsc.subcore_barrier()
    pltpu.sync_copy(accd.at[pl.ds(r0, RPT), :], degp_hbm.at[c, pl.ds(r0, RPT), :])


@functools.cache
def _get_sca():
    mesh = plsc.VectorSubcoreMesh(core_axis_name="c", subcore_axis_name="s")
    return pl.kernel(
        _sca_body,
        mesh=mesh,
        out_type=jax.ShapeDtypeStruct((NC, NP, 16), jnp.float32),
        compiler_params=pltpu.CompilerParams(use_tc_tiling_on_sc=False),
        scratch_types=[
            pltpu.VMEM((GB, GSZ), jnp.int32),
            pltpu.VMEM((GSZ, 16), jnp.float32),
            pltpu.VMEM_SHARED((NP, 16), jnp.float32),
        ],
    )


def _scb_body(y_hbm, src_hbm, dst_hbm, z_hbm, s_hbm, srci, dsti, gbuf, sem,
              acc):
    c = lax.axis_index("c")
    s = lax.axis_index("s")
    r0 = s * RPT
    pltpu.sync_copy(z_hbm.at[pl.ds(r0, RPT), :], acc.at[pl.ds(r0, RPT), :])
    plsc.subcore_barrier()
    g0_tile = s * GPT

    def outer(k, carry):
        g0 = g0_tile + k * GB
        pltpu.sync_copy(src_hbm.at[c, pl.ds(g0, GB), :], srci)
        pltpu.sync_copy(dst_hbm.at[pl.ds(g0, GB), :], dsti)
        for p in range(NBUF - 1):
            pltpu.async_copy(y_hbm.at[srci.at[p]], gbuf.at[p], sem)

        def inner(j, carry2):
            @pl.when(j + NBUF - 1 < GB)
            def _():
                pltpu.async_copy(y_hbm.at[srci.at[j + NBUF - 1]],
                                 gbuf.at[(j + NBUF - 1) % NBUF], sem)

            pltpu.make_async_copy(y_hbm.at[srci.at[j]], gbuf.at[j % NBUF],
                                  sem).wait()
            pltpu.sync_copy(gbuf.at[j % NBUF], acc.at[dsti.at[j]], add=True)
            return carry2

        return lax.fori_loop(0, GB, inner, carry)

    lax.fori_loop(0, GPT // GB, outer, 0)
    plsc.subcore_barrier()
    pltpu.sync_copy(acc.at[pl.ds(r0, RPT), :], s_hbm.at[c, pl.ds(r0, RPT), :])


@functools.cache
def _get_scb():
    mesh = plsc.VectorSubcoreMesh(core_axis_name="c", subcore_axis_name="s")
    return pl.kernel(
        _scb_body,
        mesh=mesh,
        out_type=jax.ShapeDtypeStruct((NC, NP, HD), jnp.float32),
        compiler_params=pltpu.CompilerParams(use_tc_tiling_on_sc=False),
        scratch_types=[
            pltpu.VMEM((GB, GSZ), jnp.int32),
            pltpu.VMEM((GB, GSZ), jnp.int32),
            pltpu.VMEM((NBUF, GSZ, HD), jnp.float32),
            pltpu.SemaphoreType.DMA,
            pltpu.VMEM_SHARED((NP, HD), jnp.float32),
        ],
    )


def _tc1_body(x_ref, w_ref, degp_ref, y_ref, dinv_ref):
    dp = degp_ref[...]
    deg = dp[0, :, 0] + dp[1, :, 0] + 2.0
    dinv = lax.rsqrt(deg)
    xw = jnp.dot(x_ref[...], w_ref[...], preferred_element_type=jnp.float32)
    y = xw * dinv[:, None]
    y_ref[0] = y[:, :HD]
    y_ref[1] = y[:, HD:]
    dinv_ref[...] = dinv[:, None]


_tc1 = pl.pallas_call(
    _tc1_body,
    grid=(NBLK,),
    in_specs=[
        pl.BlockSpec((BM, D), lambda i: (i, 0)),
        pl.BlockSpec((D, D), lambda i: (0, 0)),
        pl.BlockSpec((NC, BM, 16), lambda i: (0, i, 0)),
    ],
    out_specs=[
        pl.BlockSpec((NC, BM, HD), lambda i: (0, i, 0)),
        pl.BlockSpec((BM, 1), lambda i: (i, 0)),
    ],
    out_shape=[
        jax.ShapeDtypeStruct((NC, N, HD), jnp.float32),
        jax.ShapeDtypeStruct((NP, 1), jnp.float32),
    ],
)


def _tcmid_body(s_ref, y_ref, dinv_ref, b_ref, w_ref, yn_ref):
    sv = s_ref[...]
    yv = y_ref[...]
    sb = jnp.concatenate([sv[0], sv[1]], axis=1)
    yb = jnp.concatenate([yv[0], yv[1]], axis=1)
    dinv = dinv_ref[...]
    h = jnp.maximum(dinv * (sb + 2.0 * yb) + b_ref[...], 0.0)
    yn = jnp.dot(h, w_ref[...], preferred_element_type=jnp.float32) * dinv
    yn_ref[0] = yn[:, :HD]
    yn_ref[1] = yn[:, HD:]


_tcmid = pl.pallas_call(
    _tcmid_body,
    grid=(NBLK,),
    in_specs=[
        pl.BlockSpec((NC, BM, HD), lambda i: (0, i, 0)),
        pl.BlockSpec((NC, BM, HD), lambda i: (0, i, 0)),
        pl.BlockSpec((BM, 1), lambda i: (i, 0)),
        pl.BlockSpec((1, D), lambda i: (0, 0)),
        pl.BlockSpec((D, D), lambda i: (0, 0)),
    ],
    out_specs=pl.BlockSpec((NC, BM, HD), lambda i: (0, i, 0)),
    out_shape=jax.ShapeDtypeStruct((NC, N, HD), jnp.float32),
)


def _tc4_body(s_ref, y_ref, dinv_ref, b_ref, batch_ref, wout_ref, bout_ref,
              out_ref, acc_ref):
    i = pl.program_id(0)

    @pl.when(i == 0)
    def _():
        acc_ref[...] = jnp.zeros_like(acc_ref)

    sv = s_ref[...]
    yv = y_ref[...]
    sb = jnp.concatenate([sv[0], sv[1]], axis=1)
    yb = jnp.concatenate([yv[0], yv[1]], axis=1)
    dinv = dinv_ref[...]
    h = jnp.maximum(dinv * (sb + 2.0 * yb) + b_ref[...], 0.0)
    oh = (batch_ref[...] == lax.broadcasted_iota(jnp.int32, (BM, G), 1))
    acc_ref[...] += lax.dot_general(
        oh.astype(jnp.float32), h, (((0,), (0,)), ((), ())),
        preferred_element_type=jnp.float32)

    @pl.when(i == NBLK - 1)
    def _():
        out_ref[...] = (jnp.dot(acc_ref[...], wout_ref[...],
                                preferred_element_type=jnp.float32)
                        + bout_ref[...])


_tc4 = pl.pallas_call(
    _tc4_body,
    grid=(NBLK,),
    in_specs=[
        pl.BlockSpec((NC, BM, HD), lambda i: (0, i, 0)),
        pl.BlockSpec((NC, BM, HD), lambda i: (0, i, 0)),
        pl.BlockSpec((BM, 1), lambda i: (i, 0)),
        pl.BlockSpec((1, D), lambda i: (0, 0)),
        pl.BlockSpec((BM, 1), lambda i: (i, 0)),
        pl.BlockSpec((D, 1), lambda i: (0, 0)),
        pl.BlockSpec((1, 1), lambda i: (0, 0)),
    ],
    out_specs=pl.BlockSpec((G, 1), lambda i: (0, 0)),
    out_shape=jax.ShapeDtypeStruct((G, 1), jnp.float32),
    scratch_shapes=[pltpu.VMEM((G, D), jnp.float32)],
)


def kernel(x, edge_index, edge_attr, batch, W1, b1, W2, b2, Wout, bout):
    src = edge_index[0].astype(jnp.int32)
    dst = edge_index[1].astype(jnp.int32)
    pad = EP - E
    srcp = jnp.concatenate([src, jnp.zeros((pad,), jnp.int32)])
    dstp = jnp.concatenate([dst, jnp.full((pad,), N, jnp.int32)])
    src2 = jnp.stack([srcp, srcp + N]).reshape(NC, NBG, GSZ)
    dst3 = dstp.reshape(NBG, GSZ)
    zeros32 = jnp.zeros((NP, HD), jnp.float32)
    zeros16 = jnp.zeros((NP, 16), jnp.float32)
    ones16 = jnp.ones((GSZ, 16), jnp.float32)
    b1r = b1.reshape(1, D)
    b2r = b2.reshape(1, D)

    sca = _get_sca()
    scb = _get_scb()
    degp = sca(dst3, ones16, zeros16)
    y1, dinv = _tc1(x, W1, degp)
    s1 = scb(y1.reshape(NC * N, HD), src2, dst3, zeros32)
    y2 = _tcmid(s1, y1, dinv, b1r, W2)
    s2 = scb(y2.reshape(NC * N, HD), src2, dst3, zeros32)
    y3 = _tcmid(s2, y2, dinv, b2r, W2)
    s3 = scb(y3.reshape(NC * N, HD), src2, dst3, zeros32)
    return _tc4(s3, y3, dinv, b2r, batch.reshape(N, 1).astype(jnp.int32),
                Wout, bout.reshape(1, 1))
